# baseline (speedup 1.0000x reference)
import functools

import jax
import jax.numpy as jnp
import numpy as np
from jax import lax
from jax.experimental import pallas as pl
from jax.experimental.pallas import tpu as pltpu
from jax.experimental.pallas import tpu_sc as plsc

VOCAB_N = 100000
D = 1024
POS_N = 2048
BATCH = 4
B = BATCH * POS_N

_info = plsc.get_sparse_core_info()
NC, NS, LANES = _info.num_cores, _info.num_subcores, _info.num_lanes
NW = NC * NS
PB = POS_N // NW
CPP = 8
RPC = BATCH * CPP
NCH = PB // CPP
RING = 3
PREF = 2
UNROLL = 2


def _pos_table():
    half = D / 2
    positions = np.arange(POS_N, dtype=np.float32)[:, None]
    depths = np.arange(half, dtype=np.float32)[None, :] / np.float32(half)
    angle_rads = positions * (np.float32(1.0) / (10000.0 ** depths)).astype(np.float32)
    return np.concatenate([np.sin(angle_rads), np.cos(angle_rads)],
                          axis=-1).astype(np.float32)


_POS = _pos_table()


def _sc_body(table_hbm, idx_hbm, pos_hbm, out_hbm, *scr):
    idx_v = scr[0]
    bufs = scr[1:1 + RING]
    pbufs = scr[1 + RING:1 + 2 * RING]
    gsem = scr[1 + 2 * RING:1 + 3 * RING]
    psem = scr[1 + 3 * RING:1 + 4 * RING]
    ssem = scr[1 + 4 * RING:1 + 5 * RING]

    wid = lax.axis_index("s") * NC + lax.axis_index("c")
    p0 = wid * PB
    for b in range(BATCH):
        pltpu.sync_copy(idx_hbm.at[pl.ds(b * POS_N + p0, PB)],
                        idx_v.at[pl.ds(b * PB, PB)])

    def start_fetch(c):
        s = c % RING
        for b in range(BATCH):
            pltpu.async_copy(
                table_hbm.at[idx_v.at[pl.ds(b * PB + c * CPP, CPP)]],
                bufs[s].at[pl.ds(b * CPP, CPP)], gsem[s])
        pltpu.async_copy(pos_hbm.at[pl.ds(p0 + c * CPP, CPP)],
                         pbufs[s], psem[s])

    def wait_fetch(c):
        s = c % RING
        for b in range(BATCH):
            pltpu.make_async_copy(
                table_hbm.at[idx_v.at[pl.ds(b * PB + c * CPP, CPP)]],
                bufs[s].at[pl.ds(b * CPP, CPP)], gsem[s]).wait()
        pltpu.make_async_copy(pos_hbm.at[pl.ds(p0 + c * CPP, CPP)],
                              pbufs[s], psem[s]).wait()

    def start_store(c):
        s = c % RING
        for b in range(BATCH):
            pltpu.async_copy(bufs[s].at[pl.ds(b * CPP, CPP)],
                             out_hbm.at[pl.ds(b * POS_N + p0 + c * CPP, CPP)],
                             ssem[s])

    def wait_store(c):
        s = c % RING
        for b in range(BATCH):
            pltpu.make_async_copy(
                bufs[s].at[pl.ds(b * CPP, CPP)],
                out_hbm.at[pl.ds(b * POS_N + p0 + c * CPP, CPP)],
                ssem[s]).wait()

    for c in range(PREF):
        start_fetch(c)

    for c in range(NCH):
        s = c % RING
        wait_fetch(c)
        buf, pbuf = bufs[s], pbufs[s]

        def pos_loop(p, carry, buf=buf, pbuf=pbuf):
            def j_loop(j, carry2, p=p):
                for u in range(UNROLL):
                    sl = pl.ds((j * UNROLL + u) * LANES, LANES)
                    pv = pbuf[p, sl]
                    for b in range(BATCH):
                        buf[b * CPP + p, sl] = buf[b * CPP + p, sl] * 32.0 + pv
                return carry2
            return lax.fori_loop(0, D // LANES // UNROLL, j_loop, carry)

        lax.fori_loop(0, CPP, pos_loop, 0)

        start_store(c)
        cp = c + PREF
        if cp < NCH:
            if cp >= RING:
                wait_store(cp - RING)
            start_fetch(cp)

    for c in range(NCH - RING, NCH):
        wait_store(c)


@jax.jit
def _sc_embed(table, idx, pos):
    mesh = plsc.VectorSubcoreMesh(core_axis_name="c", subcore_axis_name="s")
    scratch = ([pltpu.VMEM((BATCH * PB,), jnp.int32)]
               + [pltpu.VMEM((RPC, D), jnp.float32) for _ in range(RING)]
               + [pltpu.VMEM((CPP, D), jnp.float32) for _ in range(RING)]
               + [pltpu.SemaphoreType.DMA for _ in range(3 * RING)])
    f = functools.partial(
        pl.kernel,
        mesh=mesh,
        out_type=jax.ShapeDtypeStruct((B, D), jnp.float32),
        scratch_types=scratch,
    )(_sc_body)
    return f(table, idx, pos)


def kernel(x, table):
    idx = x.reshape(-1).astype(jnp.int32)
    out = _sc_embed(table, idx, _POS)
    return out.reshape(BATCH, POS_N, D)
